# baseline (speedup 1.0000x reference)
import dataclasses
import functools

import jax
import jax.numpy as jnp
from jax import lax
from jax.experimental import pallas as pl
from jax.experimental.pallas import tpu as pltpu
from jax.experimental.pallas import tpu_sc as plsc

N = 10000
C0, C1, C2, C3 = 128, 32, 32, 64
G = 12
KT = 3
NE = 160000
T1 = G - KT + 1
T2 = T1 - 3 + 1
NTOT = N * T1
E = T1 * NE

NC = 2
NS = 16
L = 16

ACC_ROWS = 100096
EB = 128
SB = 6
EROWS = E // EB
RPT = 390
NSUP = RPT // SB
NLOOP = 10
ZCH = 368

DEG_TPT = 390
DEG_RB = 65

_mesh = plsc.VectorSubcoreMesh(core_axis_name="c", subcore_axis_name="s")

_sc_params = pltpu.CompilerParams()
if "needs_layout_passes" in pltpu.CompilerParams.__dataclass_fields__:
    _sc_params = dataclasses.replace(_sc_params, needs_layout_passes=False)
if "use_tc_tiling_on_sc" in pltpu.CompilerParams.__dataclass_fields__:
    _sc_params = dataclasses.replace(_sc_params, use_tc_tiling_on_sc=False)


@functools.partial(
    pl.kernel,
    out_type=jax.ShapeDtypeStruct((NC * NS, NTOT), jnp.float32),
    mesh=_mesh,
    compiler_params=_sc_params,
    scratch_types=[
        pltpu.VMEM((DEG_RB * EB,), jnp.int32),
        pltpu.VMEM((DEG_RB * EB,), jnp.int32),
        pltpu.VMEM((NTOT,), jnp.float32),
        pltpu.SemaphoreType.DMA,
        pltpu.SemaphoreType.DMA,
    ],
)
def _sc_degree(dst_hbm, deg_parts_hbm, dv0, dv1, hist, dsem0, dsem1):
    cid = lax.axis_index("c")
    sid = lax.axis_index("s")
    wid = sid * NC + cid
    zeros16 = jnp.zeros((L,), jnp.float32)
    ones16 = jnp.ones((L,), jnp.float32)
    dv = (dv0, dv1)
    dsem = (dsem0, dsem1)
    base = wid * DEG_TPT * EB

    def stage(p, b):
        pltpu.async_copy(dst_hbm.at[pl.ds(base + b * DEG_RB * EB,
                                          DEG_RB * EB)], dv[p], dsem[p])

    def wait_stage(p):
        pltpu.make_async_copy(dst_hbm.at[pl.ds(0, DEG_RB * EB)], dv[p],
                              dsem[p]).wait()

    def process(p, nvecs):
        @pl.loop(0, nvecs)
        def _(j):
            idx = dv[p][pl.ds(j * L, L)]
            plsc.addupdate_scatter(hist, [idx], ones16)

    @pl.loop(0, NTOT, step=L)
    def _(i):
        hist[pl.ds(i, L)] = zeros16

    stage(0, 0)

    @pl.loop(0, DEG_TPT // DEG_RB // 2)
    def _(s):
        for p in range(2):
            b = s * 2 + p
            wait_stage(p)

            @pl.when(b < DEG_TPT // DEG_RB - 1)
            def _():
                stage(1 - p, b + 1)

            process(p, DEG_RB * EB // L)

    @pl.when(wid < EROWS - 32 * DEG_TPT)
    def _():
        pltpu.sync_copy(
            dst_hbm.at[pl.ds((32 * DEG_TPT + wid) * EB, EB)],
            dv[0].at[pl.ds(0, EB)])
        process(0, EB // L)

    pltpu.sync_copy(hist, deg_parts_hbm.at[wid])


@functools.partial(
    pl.kernel,
    out_type=jax.ShapeDtypeStruct((NC, NTOT, C2), jnp.bfloat16),
    mesh=_mesh,
    compiler_params=_sc_params,
    scratch_types=[
        pltpu.VMEM((SB * EB,), jnp.int32),
        pltpu.VMEM((SB * EB,), jnp.int32),
        pltpu.VMEM((SB * EB,), jnp.int32),
        pltpu.VMEM((SB * EB,), jnp.int32),
        pltpu.VMEM((SB * EB,), jnp.int32),
        pltpu.VMEM((SB * EB,), jnp.int32),
        pltpu.VMEM((SB * EB, C2), jnp.bfloat16),
        pltpu.VMEM((SB * EB, C2), jnp.bfloat16),
        pltpu.VMEM_SHARED((ACC_ROWS, C2), jnp.bfloat16),
        pltpu.SemaphoreType.DMA,
        pltpu.SemaphoreType.DMA,
        pltpu.SemaphoreType.DMA,
        pltpu.SemaphoreType.DMA,
        pltpu.SemaphoreType.DMA,
        pltpu.SemaphoreType.DMA,
        pltpu.SemaphoreType.DMA,
    ],
)
def _sc_aggregate(src_hbm, dst_hbm, xws_hbm, agg_hbm, sx0, sx1, sx2, dx0,
                  dx1, dx2, rows0, rows1, acc, gsem0, gsem1, ssem0, ssem1,
                  isem0, isem1, isem2):
    cid = lax.axis_index("c")
    sid = lax.axis_index("s")
    zeros32 = jnp.zeros((2 * L,), jnp.bfloat16)
    wid = sid * NC + cid
    base_e = wid * RPT * EB
    sidx = (sx0, sx1, sx2)
    didx = (dx0, dx1, dx2)
    rows = (rows0, rows1)
    gsem = (gsem0, gsem1)
    ssem = (ssem0, ssem1)
    isem = (isem0, isem1, isem2)

    @pl.loop(0, ZCH)
    def _(j):
        rows0[j, pl.ds(0, 2 * L)] = zeros32

    @pl.loop(0, 17)
    def _(j):
        pltpu.sync_copy(rows0.at[pl.ds(0, ZCH)],
                        acc.at[pl.ds(sid * (17 * ZCH) + j * ZCH, ZCH)])

    plsc.subcore_barrier()

    def stage_async(i, q):
        e0 = base_e + q * SB * EB
        pltpu.async_copy(src_hbm.at[pl.ds(e0, SB * EB)], sidx[i], isem[i])
        pltpu.async_copy(dst_hbm.at[pl.ds(e0, SB * EB)], didx[i], isem[i])

    def wait_idx(i):
        pltpu.make_async_copy(src_hbm.at[pl.ds(0, SB * EB)], sidx[i],
                              isem[i]).wait()
        pltpu.make_async_copy(src_hbm.at[pl.ds(0, SB * EB)], didx[i],
                              isem[i]).wait()

    def fire_gathers(r, i):
        for k in range(SB):
            pltpu.async_copy(xws_hbm.at[sidx[i].at[pl.ds(k * EB, EB)]],
                             rows[r].at[pl.ds(k * EB, EB)], gsem[r])

    def wait_gathers(r):
        pltpu.make_async_copy(xws_hbm.at[pl.ds(0, SB * EB)], rows[r],
                              gsem[r]).wait()

    def fire_scatters(r, i):
        for k in range(SB):
            pltpu.async_copy(rows[r].at[pl.ds(k * EB, EB)],
                             acc.at[didx[i].at[pl.ds(k * EB, EB)]], ssem[r],
                             add=True)

    def wait_scatters(r):
        pltpu.make_async_copy(xws_hbm.at[pl.ds(0, SB * EB)], rows[r],
                              ssem[r]).wait()

    def phase(q, r, i, fire_next, fire_idx, guard_first):
        wait_gathers(r)
        fire_scatters(r, i)
        if fire_next:
            i1 = (i + 1) % 3
            wait_idx(i1)
            if guard_first:
                @pl.when(q >= 1)
                def _():
                    wait_scatters(1 - r)
            else:
                wait_scatters(1 - r)
            fire_gathers(1 - r, i1)
        if fire_idx:
            stage_async((i + 2) % 3, q + 2)

    pltpu.sync_copy(src_hbm.at[pl.ds(base_e, SB * EB)], sidx[0])
    pltpu.sync_copy(dst_hbm.at[pl.ds(base_e, SB * EB)], didx[0])
    fire_gathers(0, 0)
    stage_async(1, 1)

    @pl.loop(0, NLOOP)
    def _(s):
        for k in range(6):
            phase(s * 6 + k, k % 2, k % 3, True, True, k == 0)

    for q in range(6 * NLOOP, NSUP):
        phase(q, q % 2, q % 3, q < NSUP - 1, q < NSUP - 2, False)
    wait_scatters(0)
    wait_scatters(1)

    @pl.when(wid < EROWS - NC * NS * RPT)
    def _():
        e0 = (NC * NS * RPT + wid) * EB
        pltpu.sync_copy(src_hbm.at[pl.ds(e0, EB)], sidx[0].at[pl.ds(0, EB)])
        pltpu.sync_copy(dst_hbm.at[pl.ds(e0, EB)], didx[0].at[pl.ds(0, EB)])
        pltpu.sync_copy(xws_hbm.at[sidx[0].at[pl.ds(0, EB)]],
                        rows[0].at[pl.ds(0, EB)])
        pltpu.sync_copy(rows[0].at[pl.ds(0, EB)],
                        acc.at[didx[0].at[pl.ds(0, EB)]], add=True)

    plsc.subcore_barrier()

    stripe = 17 * ZCH

    @pl.when(sid < NS - 1)
    def _():
        pltpu.sync_copy(acc.at[pl.ds(sid * stripe, stripe)],
                        agg_hbm.at[cid].at[pl.ds(sid * stripe, stripe)])

    @pl.when(sid == NS - 1)
    def _():
        pltpu.sync_copy(
            acc.at[pl.ds((NS - 1) * stripe, NTOT - (NS - 1) * stripe)],
            agg_hbm.at[cid].at[pl.ds((NS - 1) * stripe,
                                     NTOT - (NS - 1) * stripe)])


def _tc12_body(x_ref, w1_ref, b1_ref, dp_ref, r_ref, wg_ref, xws_ref,
               dinv_ref):
    u = jnp.dot(x_ref[...].astype(jnp.bfloat16), w1_ref[...],
                preferred_element_type=jnp.float32) + b1_ref[...]
    a = u[:, : C1 * T1]
    g = u[:, C1 * T1:]
    h = a * jax.nn.sigmoid(g)
    deg = jnp.sum(dp_ref[...], axis=0) + 1.0
    dinv = lax.rsqrt(deg)
    dinv_e = jnp.dot(dinv, r_ref[...],
                     preferred_element_type=jnp.float32)
    xws_ref[...] = jnp.dot((h * dinv_e).astype(jnp.bfloat16), wg_ref[...],
                           preferred_element_type=jnp.float32).astype(
                               jnp.bfloat16)
    dinv_ref[...] = dinv_e


def _tc3_body(agg_ref, xws_ref, dinv_ref, bg_ref, w2_ref, b2_ref, lnw_ref,
              lnb_ref, out_ref):
    p = agg_ref[...].astype(jnp.float32)
    agg = p[0] + p[1] + xws_ref[...].astype(jnp.float32)
    pre = dinv_ref[...] * agg + bg_ref[...]
    h2 = jnp.maximum(pre, 0.0)
    u2 = jnp.dot(h2.astype(jnp.bfloat16), w2_ref[...],
                 preferred_element_type=jnp.float32) + b2_ref[...]
    a2 = u2[:, : C3 * T2]
    g2 = u2[:, C3 * T2:]
    h3 = a2 * jax.nn.sigmoid(g2)
    mu = jnp.mean(h3, axis=1, keepdims=True)
    var = jnp.mean(h3 * h3, axis=1, keepdims=True) - mu * mu
    y = (h3 - mu) * lax.rsqrt(var + 1e-5)
    out_ref[...] = y * lnw_ref[...] + lnb_ref[...]


def kernel(x, edge_index, W1, b1, Wg, bg, W2, b2, ln_w, ln_b):
    f32 = jnp.float32

    g_idx = jnp.arange(G)
    t_idx = jnp.arange(T1)
    k_idx = jnp.arange(KT)
    m1 = (g_idx[:, None, None] == t_idx[None, :, None] + k_idx[None, None, :])
    W1p = jnp.einsum("oik,gtk->igot", W1, m1.astype(f32)).reshape(
        C0 * G, 2 * C1 * T1).astype(jnp.bfloat16)
    b1p = jnp.repeat(b1, T1)

    Wg_kron = jnp.kron(jnp.eye(T1, dtype=f32), Wg).astype(jnp.bfloat16)

    tau_idx = jnp.arange(T2)
    m2 = (t_idx[:, None, None] == tau_idx[None, :, None] + k_idx[None, None, :])
    W2p = jnp.einsum("ock,tuk->ctou", W2, m2.astype(f32)).reshape(
        C2 * T1, 2 * C3 * T2).astype(jnp.bfloat16)
    b2p = jnp.repeat(b2, T2)

    bgp = jnp.tile(bg, T1)
    lnw_flat = ln_w.reshape(1, C3 * T2)
    lnb_flat = ln_b.reshape(1, C3 * T2)

    Rmat = jnp.kron(jnp.eye(T1, dtype=f32), jnp.ones((1, C2), f32))

    x2 = x.reshape(N, C0 * G)
    src = edge_index[0]
    dst = edge_index[1]

    deg_parts = _sc_degree(dst)

    NB = 400
    grid1 = N // NB

    xws, dinv_e = pl.pallas_call(
        _tc12_body,
        grid=(grid1,),
        in_specs=[
            pl.BlockSpec((NB, C0 * G), lambda i: (i, 0)),
            pl.BlockSpec((C0 * G, 2 * C1 * T1), lambda i: (0, 0)),
            pl.BlockSpec((1, 2 * C1 * T1), lambda i: (0, 0)),
            pl.BlockSpec((NC * NS, NB, T1), lambda i: (0, i, 0)),
            pl.BlockSpec((T1, C2 * T1), lambda i: (0, 0)),
            pl.BlockSpec((C1 * T1, C1 * T1), lambda i: (0, 0)),
        ],
        out_specs=[
            pl.BlockSpec((NB, C2 * T1), lambda i: (i, 0)),
            pl.BlockSpec((NB, C2 * T1), lambda i: (i, 0)),
        ],
        out_shape=[
            jax.ShapeDtypeStruct((N, C2 * T1), jnp.bfloat16),
            jax.ShapeDtypeStruct((N, C2 * T1), f32),
        ],
    )(x2, W1p, b1p.reshape(1, -1), deg_parts.reshape(NC * NS, N, T1), Rmat,
      Wg_kron)

    agg = _sc_aggregate(src, dst, xws.reshape(NTOT, C2))

    out = pl.pallas_call(
        _tc3_body,
        grid=(grid1,),
        in_specs=[
            pl.BlockSpec((NC, NB, C2 * T1), lambda i: (0, i, 0)),
            pl.BlockSpec((NB, C2 * T1), lambda i: (i, 0)),
            pl.BlockSpec((NB, C2 * T1), lambda i: (i, 0)),
            pl.BlockSpec((1, C2 * T1), lambda i: (0, 0)),
            pl.BlockSpec((C2 * T1, 2 * C3 * T2), lambda i: (0, 0)),
            pl.BlockSpec((1, 2 * C3 * T2), lambda i: (0, 0)),
            pl.BlockSpec((1, C3 * T2), lambda i: (0, 0)),
            pl.BlockSpec((1, C3 * T2), lambda i: (0, 0)),
        ],
        out_specs=pl.BlockSpec((NB, C3 * T2), lambda i: (i, 0)),
        out_shape=jax.ShapeDtypeStruct((N, C3 * T2), f32),
    )(agg.reshape(NC, N, C2 * T1), xws, dinv_e, bgp.reshape(1, -1), W2p,
      b2p.reshape(1, -1), lnw_flat, lnb_flat)

    return out.reshape(N, C3, T2)


# speedup vs baseline: 1.2928x; 1.2928x over previous
import dataclasses
import functools

import jax
import jax.numpy as jnp
from jax import lax
from jax.experimental import pallas as pl
from jax.experimental.pallas import tpu as pltpu
from jax.experimental.pallas import tpu_sc as plsc

N = 10000
C0, C1, C2, C3 = 128, 32, 32, 64
G = 12
KT = 3
NE = 160000
T1 = G - KT + 1
T2 = T1 - 3 + 1
NTOT = N * T1
E = T1 * NE

NC = 2
NS = 16
L = 16

ACC_ROWS = 100096
EB = 128
SB = 6
EROWS = E // EB
RPT = 390
NSUP = RPT // SB
NLOOP = 10
ZCH = 368

DEG_TPT = 390
DEG_RB = 65
NB = 400

_mesh = plsc.VectorSubcoreMesh(core_axis_name="c", subcore_axis_name="s")

_sc_params = pltpu.CompilerParams()
if "needs_layout_passes" in pltpu.CompilerParams.__dataclass_fields__:
    _sc_params = dataclasses.replace(_sc_params, needs_layout_passes=False)
if "use_tc_tiling_on_sc" in pltpu.CompilerParams.__dataclass_fields__:
    _sc_params = dataclasses.replace(_sc_params, use_tc_tiling_on_sc=False)


@functools.partial(
    pl.kernel,
    out_type=jax.ShapeDtypeStruct((N // NB, NC * NS, T1, NB), jnp.float32),
    mesh=_mesh,
    compiler_params=_sc_params,
    scratch_types=[
        pltpu.VMEM((DEG_RB * EB,), jnp.int32),
        pltpu.VMEM((DEG_RB * EB,), jnp.int32),
        pltpu.VMEM((NTOT,), jnp.float32),
        pltpu.SemaphoreType.DMA,
        pltpu.SemaphoreType.DMA,
    ],
)
def _sc_degree(ei_hbm, deg_parts_hbm, dv0, dv1, hist, dsem0, dsem1):
    cid = lax.axis_index("c")
    sid = lax.axis_index("s")
    wid = sid * NC + cid
    zeros16 = jnp.zeros((L,), jnp.float32)
    ones16 = jnp.ones((L,), jnp.float32)
    dv = (dv0, dv1)
    dsem = (dsem0, dsem1)
    base = E + wid * DEG_TPT * EB

    def stage(p, b):
        pltpu.async_copy(ei_hbm.at[pl.ds(base + b * DEG_RB * EB,
                                         DEG_RB * EB)], dv[p], dsem[p])

    def wait_stage(p):
        pltpu.make_async_copy(ei_hbm.at[pl.ds(0, DEG_RB * EB)], dv[p],
                              dsem[p]).wait()

    def process(p, nvecs):
        @pl.loop(0, nvecs)
        def _(j):
            idx = dv[p][pl.ds(j * L, L)]
            idx = (idx % T1) * N + idx // T1
            plsc.addupdate_scatter(hist, [idx], ones16)

    @pl.loop(0, NTOT, step=L)
    def _(i):
        hist[pl.ds(i, L)] = zeros16

    stage(0, 0)

    @pl.loop(0, DEG_TPT // DEG_RB // 2)
    def _(s):
        for p in range(2):
            b = s * 2 + p
            wait_stage(p)

            @pl.when(b < DEG_TPT // DEG_RB - 1)
            def _():
                stage(1 - p, b + 1)

            process(p, DEG_RB * EB // L)

    @pl.when(wid < EROWS - 32 * DEG_TPT)
    def _():
        pltpu.sync_copy(
            ei_hbm.at[pl.ds(E + (32 * DEG_TPT + wid) * EB, EB)],
            dv[0].at[pl.ds(0, EB)])
        process(0, EB // L)

    @pl.loop(0, N // NB)
    def _(j):
        for k in range(T1):
            pltpu.async_copy(hist.at[pl.ds(k * N + j * NB, NB)],
                             deg_parts_hbm.at[j].at[wid].at[k], dsem0)

    @pl.loop(0, (N // NB) * T1)
    def _(j):
        pltpu.make_async_copy(hist.at[pl.ds(0, NB)],
                              deg_parts_hbm.at[0].at[wid].at[0],
                              dsem0).wait()


@functools.partial(
    pl.kernel,
    out_type=jax.ShapeDtypeStruct((NC, NTOT, C2), jnp.bfloat16),
    mesh=_mesh,
    compiler_params=_sc_params,
    scratch_types=[
        pltpu.VMEM((SB * EB,), jnp.int32),
        pltpu.VMEM((SB * EB,), jnp.int32),
        pltpu.VMEM((SB * EB,), jnp.int32),
        pltpu.VMEM((SB * EB,), jnp.int32),
        pltpu.VMEM((SB * EB,), jnp.int32),
        pltpu.VMEM((SB * EB,), jnp.int32),
        pltpu.VMEM((SB * EB, C2), jnp.bfloat16),
        pltpu.VMEM((SB * EB, C2), jnp.bfloat16),
        pltpu.VMEM_SHARED((ACC_ROWS, C2), jnp.bfloat16),
        pltpu.SemaphoreType.DMA,
        pltpu.SemaphoreType.DMA,
        pltpu.SemaphoreType.DMA,
        pltpu.SemaphoreType.DMA,
        pltpu.SemaphoreType.DMA,
        pltpu.SemaphoreType.DMA,
        pltpu.SemaphoreType.DMA,
    ],
)
def _sc_aggregate(ei_hbm, xws_hbm, agg_hbm, sx0, sx1, sx2, dx0,
                  dx1, dx2, rows0, rows1, acc, gsem0, gsem1, ssem0, ssem1,
                  isem0, isem1, isem2):
    cid = lax.axis_index("c")
    sid = lax.axis_index("s")
    zeros32 = jnp.zeros((2 * L,), jnp.bfloat16)
    wid = sid * NC + cid
    base_e = wid * RPT * EB
    sidx = (sx0, sx1, sx2)
    didx = (dx0, dx1, dx2)
    rows = (rows0, rows1)
    gsem = (gsem0, gsem1)
    ssem = (ssem0, ssem1)
    isem = (isem0, isem1, isem2)

    @pl.loop(0, ZCH)
    def _(j):
        rows0[j, pl.ds(0, 2 * L)] = zeros32

    @pl.loop(0, 17)
    def _(j):
        pltpu.sync_copy(rows0.at[pl.ds(0, ZCH)],
                        acc.at[pl.ds(sid * (17 * ZCH) + j * ZCH, ZCH)])

    plsc.subcore_barrier()

    def stage_async(i, q):
        e0 = base_e + q * SB * EB
        pltpu.async_copy(ei_hbm.at[pl.ds(e0, SB * EB)], sidx[i], isem[i])
        pltpu.async_copy(ei_hbm.at[pl.ds(E + e0, SB * EB)], didx[i], isem[i])

    def wait_idx(i):
        pltpu.make_async_copy(ei_hbm.at[pl.ds(0, SB * EB)], sidx[i],
                              isem[i]).wait()
        pltpu.make_async_copy(ei_hbm.at[pl.ds(0, SB * EB)], didx[i],
                              isem[i]).wait()

    def fire_gathers(r, i):
        for k in range(SB):
            pltpu.async_copy(xws_hbm.at[sidx[i].at[pl.ds(k * EB, EB)]],
                             rows[r].at[pl.ds(k * EB, EB)], gsem[r])

    def wait_gathers(r):
        pltpu.make_async_copy(xws_hbm.at[pl.ds(0, SB * EB)], rows[r],
                              gsem[r]).wait()

    def fire_scatters(r, i):
        for k in range(SB):
            pltpu.async_copy(rows[r].at[pl.ds(k * EB, EB)],
                             acc.at[didx[i].at[pl.ds(k * EB, EB)]], ssem[r],
                             add=True)

    def wait_scatters(r):
        pltpu.make_async_copy(xws_hbm.at[pl.ds(0, SB * EB)], rows[r],
                              ssem[r]).wait()

    def phase(q, r, i, fire_next, fire_idx, guard_first):
        wait_gathers(r)
        fire_scatters(r, i)
        if fire_next:
            i1 = (i + 1) % 3
            wait_idx(i1)
            if guard_first:
                @pl.when(q >= 1)
                def _():
                    wait_scatters(1 - r)
            else:
                wait_scatters(1 - r)
            fire_gathers(1 - r, i1)
        if fire_idx:
            stage_async((i + 2) % 3, q + 2)

    pltpu.sync_copy(ei_hbm.at[pl.ds(base_e, SB * EB)], sidx[0])
    pltpu.sync_copy(ei_hbm.at[pl.ds(E + base_e, SB * EB)], didx[0])
    fire_gathers(0, 0)
    stage_async(1, 1)

    @pl.loop(0, NLOOP)
    def _(s):
        for k in range(6):
            phase(s * 6 + k, k % 2, k % 3, True, True, k == 0)

    for q in range(6 * NLOOP, NSUP):
        phase(q, q % 2, q % 3, q < NSUP - 1, q < NSUP - 2, False)
    wait_scatters(0)
    wait_scatters(1)

    @pl.when(wid < EROWS - NC * NS * RPT)
    def _():
        e0 = (NC * NS * RPT + wid) * EB
        pltpu.sync_copy(ei_hbm.at[pl.ds(e0, EB)], sidx[0].at[pl.ds(0, EB)])
        pltpu.sync_copy(ei_hbm.at[pl.ds(E + e0, EB)],
                        didx[0].at[pl.ds(0, EB)])
        pltpu.sync_copy(xws_hbm.at[sidx[0].at[pl.ds(0, EB)]],
                        rows[0].at[pl.ds(0, EB)])
        pltpu.sync_copy(rows[0].at[pl.ds(0, EB)],
                        acc.at[didx[0].at[pl.ds(0, EB)]], add=True)

    plsc.subcore_barrier()

    stripe = 17 * ZCH

    @pl.when(sid < NS - 1)
    def _():
        pltpu.sync_copy(acc.at[pl.ds(sid * stripe, stripe)],
                        agg_hbm.at[cid].at[pl.ds(sid * stripe, stripe)])

    @pl.when(sid == NS - 1)
    def _():
        pltpu.sync_copy(
            acc.at[pl.ds((NS - 1) * stripe, NTOT - (NS - 1) * stripe)],
            agg_hbm.at[cid].at[pl.ds((NS - 1) * stripe,
                                     NTOT - (NS - 1) * stripe)])


def _tc12_body(x_ref, w1_ref, b1_ref, dp_ref, r_ref, wg_ref, xws_ref,
               dinv_ref):
    u = jnp.dot(x_ref[...].astype(jnp.bfloat16), w1_ref[...],
                preferred_element_type=jnp.float32) + b1_ref[...]
    a = u[:, : C1 * T1]
    g = u[:, C1 * T1:]
    h = a * jax.nn.sigmoid(g)
    degT = jnp.sum(dp_ref[0], axis=0) + 1.0
    dinvT = lax.rsqrt(degT)
    dinv_e = lax.dot_general(dinvT, r_ref[...], (((0,), (0,)), ((), ())),
                             preferred_element_type=jnp.float32)
    xws = jnp.dot((h * dinv_e).astype(jnp.bfloat16), wg_ref[...],
                  preferred_element_type=jnp.float32)
    xws_ref[...] = xws.astype(jnp.bfloat16)
    dinv_ref[...] = dinv_e


def _tc3_body(agg_ref, xws_ref, dinv_ref, bg_ref, w2_ref, b2_ref, lnw_ref,
              lnb_ref, out_ref):
    p = agg_ref[...].astype(jnp.float32)
    agg = p[0] + p[1] + xws_ref[...].astype(jnp.float32)
    pre = dinv_ref[...] * agg + bg_ref[...]
    h2 = jnp.maximum(pre, 0.0)
    u2 = jnp.dot(h2.astype(jnp.bfloat16), w2_ref[...],
                 preferred_element_type=jnp.float32) + b2_ref[...]
    a2 = u2[:, : C3 * T2]
    g2 = u2[:, C3 * T2:]
    h3 = a2 * jax.nn.sigmoid(g2)
    mu = jnp.mean(h3, axis=1, keepdims=True)
    var = jnp.mean(h3 * h3, axis=1, keepdims=True) - mu * mu
    y = (h3 - mu) * lax.rsqrt(var + 1e-5)
    out_ref[...] = y * lnw_ref[...] + lnb_ref[...]


def kernel(x, edge_index, W1, b1, Wg, bg, W2, b2, ln_w, ln_b):
    f32 = jnp.float32

    g_idx = jnp.arange(G)
    t_idx = jnp.arange(T1)
    k_idx = jnp.arange(KT)
    m1 = (g_idx[:, None, None] == t_idx[None, :, None] + k_idx[None, None, :])
    W1p = jnp.einsum("oik,gtk->igot", W1, m1.astype(f32)).reshape(
        C0 * G, 2 * C1 * T1).astype(jnp.bfloat16)
    b1p = jnp.repeat(b1, T1)

    Wg_kron = jnp.kron(jnp.eye(T1, dtype=f32), Wg).astype(jnp.bfloat16)

    tau_idx = jnp.arange(T2)
    m2 = (t_idx[:, None, None] == tau_idx[None, :, None] + k_idx[None, None, :])
    W2p = jnp.einsum("ock,tuk->ctou", W2, m2.astype(f32)).reshape(
        C2 * T1, 2 * C3 * T2).astype(jnp.bfloat16)
    b2p = jnp.repeat(b2, T2)

    bgp = jnp.tile(bg, T1)
    lnw_flat = ln_w.reshape(1, C3 * T2)
    lnb_flat = ln_b.reshape(1, C3 * T2)

    Rmat = jnp.kron(jnp.eye(T1, dtype=f32), jnp.ones((1, C2), f32))

    eiflat = edge_index.reshape(2 * E)

    deg_parts = _sc_degree(eiflat)

    grid1 = N // NB

    xws, dinv_e = pl.pallas_call(
        _tc12_body,
        grid=(grid1,),
        in_specs=[
            pl.BlockSpec((NB, C0 * G), lambda i: (i, 0)),
            pl.BlockSpec((C0 * G, 2 * C1 * T1), lambda i: (0, 0)),
            pl.BlockSpec((1, 2 * C1 * T1), lambda i: (0, 0)),
            pl.BlockSpec((1, NC * NS, T1, NB), lambda i: (i, 0, 0, 0)),
            pl.BlockSpec((T1, C2 * T1), lambda i: (0, 0)),
            pl.BlockSpec((C1 * T1, C1 * T1), lambda i: (0, 0)),
        ],
        out_specs=[
            pl.BlockSpec((NB, C2 * T1), lambda i: (i, 0)),
            pl.BlockSpec((NB, C2 * T1), lambda i: (i, 0)),
        ],
        out_shape=[
            jax.ShapeDtypeStruct((N, C2 * T1), jnp.bfloat16),
            jax.ShapeDtypeStruct((N, C2 * T1), f32),
        ],
    )(x.reshape(N, C0 * G), W1p, b1p.reshape(1, -1), deg_parts, Rmat,
      Wg_kron)

    agg = _sc_aggregate(eiflat, xws.reshape(NTOT, C2))

    out = pl.pallas_call(
        _tc3_body,
        grid=(grid1,),
        in_specs=[
            pl.BlockSpec((NC, NB, C2 * T1), lambda i: (0, i, 0)),
            pl.BlockSpec((NB, C2 * T1), lambda i: (i, 0)),
            pl.BlockSpec((NB, C2 * T1), lambda i: (i, 0)),
            pl.BlockSpec((1, C2 * T1), lambda i: (0, 0)),
            pl.BlockSpec((C2 * T1, 2 * C3 * T2), lambda i: (0, 0)),
            pl.BlockSpec((1, 2 * C3 * T2), lambda i: (0, 0)),
            pl.BlockSpec((1, C3 * T2), lambda i: (0, 0)),
            pl.BlockSpec((1, C3 * T2), lambda i: (0, 0)),
        ],
        out_specs=pl.BlockSpec((NB, C3 * T2), lambda i: (i, 0)),
        out_shape=jax.ShapeDtypeStruct((N, C3 * T2), f32),
    )(agg.reshape(NC, N, C2 * T1), xws, dinv_e, bgp.reshape(1, -1), W2p,
      b2p.reshape(1, -1), lnw_flat, lnb_flat)

    return out.reshape(N, C3, T2)


# speedup vs baseline: 1.2989x; 1.0047x over previous
import dataclasses
import functools

import jax
import jax.numpy as jnp
from jax import lax
from jax.experimental import pallas as pl
from jax.experimental.pallas import tpu as pltpu
from jax.experimental.pallas import tpu_sc as plsc

N = 10000
C0, C1, C2, C3 = 128, 32, 32, 64
G = 12
KT = 3
NE = 160000
T1 = G - KT + 1
T2 = T1 - 3 + 1
NTOT = N * T1
E = T1 * NE

NC = 2
NS = 16
L = 16

ACC_ROWS = 100096
EB = 128
SB = 6
EROWS = E // EB
RPT = 390
NSUP = RPT // SB
NLOOP = 10
ZCH = 368

DEG_TPT = 390
DEG_RB = 65
NB = 400

_mesh = plsc.VectorSubcoreMesh(core_axis_name="c", subcore_axis_name="s")

_sc_params = pltpu.CompilerParams()
if "needs_layout_passes" in pltpu.CompilerParams.__dataclass_fields__:
    _sc_params = dataclasses.replace(_sc_params, needs_layout_passes=False)
if "use_tc_tiling_on_sc" in pltpu.CompilerParams.__dataclass_fields__:
    _sc_params = dataclasses.replace(_sc_params, use_tc_tiling_on_sc=False)


@functools.partial(
    pl.kernel,
    out_type=jax.ShapeDtypeStruct((N // NB, NC * NS, T1, NB), jnp.float32),
    mesh=_mesh,
    compiler_params=_sc_params,
    scratch_types=[
        pltpu.VMEM((DEG_RB * EB,), jnp.int32),
        pltpu.VMEM((DEG_RB * EB,), jnp.int32),
        pltpu.VMEM((NTOT,), jnp.float32),
        pltpu.SemaphoreType.DMA,
        pltpu.SemaphoreType.DMA,
    ],
)
def _sc_degree(ei_hbm, deg_parts_hbm, dv0, dv1, hist, dsem0, dsem1):
    cid = lax.axis_index("c")
    sid = lax.axis_index("s")
    wid = sid * NC + cid
    zeros16 = jnp.zeros((L,), jnp.float32)
    ones16 = jnp.ones((L,), jnp.float32)
    dv = (dv0, dv1)
    dsem = (dsem0, dsem1)
    base = E + wid * DEG_TPT * EB

    def stage(p, b):
        pltpu.async_copy(ei_hbm.at[pl.ds(base + b * DEG_RB * EB,
                                         DEG_RB * EB)], dv[p], dsem[p])

    def wait_stage(p):
        pltpu.make_async_copy(ei_hbm.at[pl.ds(0, DEG_RB * EB)], dv[p],
                              dsem[p]).wait()

    def process(p, nvecs):
        @pl.loop(0, nvecs)
        def _(j):
            idx = dv[p][pl.ds(j * L, L)]
            q = (idx.astype(jnp.float32) *
                 jnp.float32(0.1)).astype(jnp.int32)
            idx = (idx - q * T1) * N + q
            plsc.addupdate_scatter(hist, [idx], ones16)

    @pl.loop(0, NTOT, step=L)
    def _(i):
        hist[pl.ds(i, L)] = zeros16

    stage(0, 0)

    @pl.loop(0, DEG_TPT // DEG_RB // 2)
    def _(s):
        for p in range(2):
            b = s * 2 + p
            wait_stage(p)

            @pl.when(b < DEG_TPT // DEG_RB - 1)
            def _():
                stage(1 - p, b + 1)

            process(p, DEG_RB * EB // L)

    @pl.when(wid < EROWS - 32 * DEG_TPT)
    def _():
        pltpu.sync_copy(
            ei_hbm.at[pl.ds(E + (32 * DEG_TPT + wid) * EB, EB)],
            dv[0].at[pl.ds(0, EB)])
        process(0, EB // L)

    @pl.loop(0, N // NB)
    def _(j):
        for k in range(T1):
            pltpu.async_copy(hist.at[pl.ds(k * N + j * NB, NB)],
                             deg_parts_hbm.at[j].at[wid].at[k], dsem0)

    @pl.loop(0, (N // NB) * T1)
    def _(j):
        pltpu.make_async_copy(hist.at[pl.ds(0, NB)],
                              deg_parts_hbm.at[0].at[wid].at[0],
                              dsem0).wait()


@functools.partial(
    pl.kernel,
    out_type=jax.ShapeDtypeStruct((NC, NTOT, C2), jnp.bfloat16),
    mesh=_mesh,
    compiler_params=_sc_params,
    scratch_types=[
        pltpu.VMEM((SB * EB,), jnp.int32),
        pltpu.VMEM((SB * EB,), jnp.int32),
        pltpu.VMEM((SB * EB,), jnp.int32),
        pltpu.VMEM((SB * EB,), jnp.int32),
        pltpu.VMEM((SB * EB,), jnp.int32),
        pltpu.VMEM((SB * EB,), jnp.int32),
        pltpu.VMEM((SB * EB, C2), jnp.bfloat16),
        pltpu.VMEM((SB * EB, C2), jnp.bfloat16),
        pltpu.VMEM_SHARED((ACC_ROWS, C2), jnp.bfloat16),
        pltpu.SemaphoreType.DMA,
        pltpu.SemaphoreType.DMA,
        pltpu.SemaphoreType.DMA,
        pltpu.SemaphoreType.DMA,
        pltpu.SemaphoreType.DMA,
        pltpu.SemaphoreType.DMA,
        pltpu.SemaphoreType.DMA,
    ],
)
def _sc_aggregate(ei_hbm, xws_hbm, agg_hbm, sx0, sx1, sx2, dx0,
                  dx1, dx2, rows0, rows1, acc, gsem0, gsem1, ssem0, ssem1,
                  isem0, isem1, isem2):
    cid = lax.axis_index("c")
    sid = lax.axis_index("s")
    zeros32 = jnp.zeros((2 * L,), jnp.bfloat16)
    wid = sid * NC + cid
    base_e = wid * RPT * EB
    sidx = (sx0, sx1, sx2)
    didx = (dx0, dx1, dx2)
    rows = (rows0, rows1)
    gsem = (gsem0, gsem1)
    ssem = (ssem0, ssem1)
    isem = (isem0, isem1, isem2)

    @pl.loop(0, ZCH)
    def _(j):
        rows0[j, pl.ds(0, 2 * L)] = zeros32

    @pl.loop(0, 17)
    def _(j):
        pltpu.sync_copy(rows0.at[pl.ds(0, ZCH)],
                        acc.at[pl.ds(sid * (17 * ZCH) + j * ZCH, ZCH)])

    plsc.subcore_barrier()

    def stage_async(i, q):
        e0 = base_e + q * SB * EB
        pltpu.async_copy(ei_hbm.at[pl.ds(e0, SB * EB)], sidx[i], isem[i])
        pltpu.async_copy(ei_hbm.at[pl.ds(E + e0, SB * EB)], didx[i], isem[i])

    def wait_idx(i):
        pltpu.make_async_copy(ei_hbm.at[pl.ds(0, SB * EB)], sidx[i],
                              isem[i]).wait()
        pltpu.make_async_copy(ei_hbm.at[pl.ds(0, SB * EB)], didx[i],
                              isem[i]).wait()

    def fire_gathers(r, i):
        for k in range(SB):
            pltpu.async_copy(xws_hbm.at[sidx[i].at[pl.ds(k * EB, EB)]],
                             rows[r].at[pl.ds(k * EB, EB)], gsem[r])

    def wait_gathers(r):
        pltpu.make_async_copy(xws_hbm.at[pl.ds(0, SB * EB)], rows[r],
                              gsem[r]).wait()

    def fire_scatters(r, i):
        for k in range(SB):
            pltpu.async_copy(rows[r].at[pl.ds(k * EB, EB)],
                             acc.at[didx[i].at[pl.ds(k * EB, EB)]], ssem[r],
                             add=True)

    def wait_scatters(r):
        pltpu.make_async_copy(xws_hbm.at[pl.ds(0, SB * EB)], rows[r],
                              ssem[r]).wait()

    def phase(q, r, i, fire_next, fire_idx, guard_first):
        wait_gathers(r)
        fire_scatters(r, i)
        if fire_next:
            i1 = (i + 1) % 3
            wait_idx(i1)
            if guard_first:
                @pl.when(q >= 1)
                def _():
                    wait_scatters(1 - r)
            else:
                wait_scatters(1 - r)
            fire_gathers(1 - r, i1)
        if fire_idx:
            stage_async((i + 2) % 3, q + 2)

    pltpu.sync_copy(ei_hbm.at[pl.ds(base_e, SB * EB)], sidx[0])
    pltpu.sync_copy(ei_hbm.at[pl.ds(E + base_e, SB * EB)], didx[0])
    fire_gathers(0, 0)
    stage_async(1, 1)

    @pl.loop(0, NLOOP)
    def _(s):
        for k in range(6):
            phase(s * 6 + k, k % 2, k % 3, True, True, k == 0)

    for q in range(6 * NLOOP, NSUP):
        phase(q, q % 2, q % 3, q < NSUP - 1, q < NSUP - 2, False)
    wait_scatters(0)
    wait_scatters(1)

    @pl.when(wid < EROWS - NC * NS * RPT)
    def _():
        e0 = (NC * NS * RPT + wid) * EB
        pltpu.sync_copy(ei_hbm.at[pl.ds(e0, EB)], sidx[0].at[pl.ds(0, EB)])
        pltpu.sync_copy(ei_hbm.at[pl.ds(E + e0, EB)],
                        didx[0].at[pl.ds(0, EB)])
        pltpu.sync_copy(xws_hbm.at[sidx[0].at[pl.ds(0, EB)]],
                        rows[0].at[pl.ds(0, EB)])
        pltpu.sync_copy(rows[0].at[pl.ds(0, EB)],
                        acc.at[didx[0].at[pl.ds(0, EB)]], add=True)

    plsc.subcore_barrier()

    stripe = 17 * ZCH

    @pl.when(sid < NS - 1)
    def _():
        pltpu.sync_copy(acc.at[pl.ds(sid * stripe, stripe)],
                        agg_hbm.at[cid].at[pl.ds(sid * stripe, stripe)])

    @pl.when(sid == NS - 1)
    def _():
        pltpu.sync_copy(
            acc.at[pl.ds((NS - 1) * stripe, NTOT - (NS - 1) * stripe)],
            agg_hbm.at[cid].at[pl.ds((NS - 1) * stripe,
                                     NTOT - (NS - 1) * stripe)])


def _tc12_body(x_ref, w1_ref, b1_ref, dp_ref, r_ref, wg_ref, xws_ref,
               dinv_ref):
    u = jnp.dot(x_ref[...].astype(jnp.bfloat16), w1_ref[...],
                preferred_element_type=jnp.float32) + b1_ref[...]
    a = u[:, : C1 * T1]
    g = u[:, C1 * T1:]
    h = a * jax.nn.sigmoid(g)
    degT = jnp.sum(dp_ref[0], axis=0) + 1.0
    dinvT = lax.rsqrt(degT)
    dinv_e = lax.dot_general(dinvT, r_ref[...], (((0,), (0,)), ((), ())),
                             preferred_element_type=jnp.float32)
    xws = jnp.dot((h * dinv_e).astype(jnp.bfloat16), wg_ref[...],
                  preferred_element_type=jnp.float32)
    xws_ref[...] = xws.astype(jnp.bfloat16)
    dinv_ref[...] = dinv_e


def _tc3_body(agg_ref, xws_ref, dinv_ref, bg_ref, w2_ref, b2_ref, lnw_ref,
              lnb_ref, out_ref):
    p = agg_ref[...].astype(jnp.float32)
    agg = p[0] + p[1] + xws_ref[...].astype(jnp.float32)
    pre = dinv_ref[...] * agg + bg_ref[...]
    h2 = jnp.maximum(pre, 0.0)
    u2 = jnp.dot(h2.astype(jnp.bfloat16), w2_ref[...],
                 preferred_element_type=jnp.float32) + b2_ref[...]
    a2 = u2[:, : C3 * T2]
    g2 = u2[:, C3 * T2:]
    h3 = a2 * jax.nn.sigmoid(g2)
    mu = jnp.mean(h3, axis=1, keepdims=True)
    var = jnp.mean(h3 * h3, axis=1, keepdims=True) - mu * mu
    y = (h3 - mu) * lax.rsqrt(var + 1e-5)
    out_ref[...] = y * lnw_ref[...] + lnb_ref[...]


def kernel(x, edge_index, W1, b1, Wg, bg, W2, b2, ln_w, ln_b):
    f32 = jnp.float32

    g_idx = jnp.arange(G)
    t_idx = jnp.arange(T1)
    k_idx = jnp.arange(KT)
    m1 = (g_idx[:, None, None] == t_idx[None, :, None] + k_idx[None, None, :])
    W1p = jnp.einsum("oik,gtk->igot", W1, m1.astype(f32)).reshape(
        C0 * G, 2 * C1 * T1).astype(jnp.bfloat16)
    b1p = jnp.repeat(b1, T1)

    Wg_kron = jnp.kron(jnp.eye(T1, dtype=f32), Wg).astype(jnp.bfloat16)

    tau_idx = jnp.arange(T2)
    m2 = (t_idx[:, None, None] == tau_idx[None, :, None] + k_idx[None, None, :])
    W2p = jnp.einsum("ock,tuk->ctou", W2, m2.astype(f32)).reshape(
        C2 * T1, 2 * C3 * T2).astype(jnp.bfloat16)
    b2p = jnp.repeat(b2, T2)

    bgp = jnp.tile(bg, T1)
    lnw_flat = ln_w.reshape(1, C3 * T2)
    lnb_flat = ln_b.reshape(1, C3 * T2)

    Rmat = jnp.kron(jnp.eye(T1, dtype=f32), jnp.ones((1, C2), f32))

    eiflat = edge_index.reshape(2 * E)

    deg_parts = _sc_degree(eiflat)

    grid1 = N // NB

    xws, dinv_e = pl.pallas_call(
        _tc12_body,
        grid=(grid1,),
        in_specs=[
            pl.BlockSpec((NB, C0 * G), lambda i: (i, 0)),
            pl.BlockSpec((C0 * G, 2 * C1 * T1), lambda i: (0, 0)),
            pl.BlockSpec((1, 2 * C1 * T1), lambda i: (0, 0)),
            pl.BlockSpec((1, NC * NS, T1, NB), lambda i: (i, 0, 0, 0)),
            pl.BlockSpec((T1, C2 * T1), lambda i: (0, 0)),
            pl.BlockSpec((C1 * T1, C1 * T1), lambda i: (0, 0)),
        ],
        out_specs=[
            pl.BlockSpec((NB, C2 * T1), lambda i: (i, 0)),
            pl.BlockSpec((NB, C2 * T1), lambda i: (i, 0)),
        ],
        out_shape=[
            jax.ShapeDtypeStruct((N, C2 * T1), jnp.bfloat16),
            jax.ShapeDtypeStruct((N, C2 * T1), f32),
        ],
    )(x.reshape(N, C0 * G), W1p, b1p.reshape(1, -1), deg_parts, Rmat,
      Wg_kron)

    agg = _sc_aggregate(eiflat, xws.reshape(NTOT, C2))

    out = pl.pallas_call(
        _tc3_body,
        grid=(grid1,),
        in_specs=[
            pl.BlockSpec((NC, NB, C2 * T1), lambda i: (0, i, 0)),
            pl.BlockSpec((NB, C2 * T1), lambda i: (i, 0)),
            pl.BlockSpec((NB, C2 * T1), lambda i: (i, 0)),
            pl.BlockSpec((1, C2 * T1), lambda i: (0, 0)),
            pl.BlockSpec((C2 * T1, 2 * C3 * T2), lambda i: (0, 0)),
            pl.BlockSpec((1, 2 * C3 * T2), lambda i: (0, 0)),
            pl.BlockSpec((1, C3 * T2), lambda i: (0, 0)),
            pl.BlockSpec((1, C3 * T2), lambda i: (0, 0)),
        ],
        out_specs=pl.BlockSpec((NB, C3 * T2), lambda i: (i, 0)),
        out_shape=jax.ShapeDtypeStruct((N, C3 * T2), f32),
    )(agg.reshape(NC, N, C2 * T1), xws, dinv_e, bgp.reshape(1, -1), W2p,
      b2p.reshape(1, -1), lnw_flat, lnb_flat)

    return out.reshape(N, C3, T2)
